# Initial kernel scaffold; baseline (speedup 1.0000x reference)
import jax
import jax.numpy as jnp
from jax.experimental import pallas as pl


def kernel(x, edge_index, batch, W1, b1, Ws1, bs1, W2, b2, Ws2, bs2, W3, b3, Ws3, bs3, L1w, L1b, L2w, L2b, L3w, L3b):
    raise NotImplementedError("write your pallas kernel here")


# speedup vs baseline: 36.8915x; 36.8915x over previous
import functools

import jax
import jax.numpy as jnp
from jax import lax
from jax.experimental import pallas as pl
from jax.experimental.pallas import tpu as pltpu
from jax.experimental.pallas import tpu_sc as plsc

N = 10000
E = 320000
B = 64
H = 128
NP = 10240
DUMMY = 10208
NC, NS, L = 2, 16, 16
NW = NC * NS
EPW = E // NW
EC = 400
RPT = NP // NS
NWL = NW * L
ECM = 200
NEG = -2147483648

_mesh = plsc.VectorSubcoreMesh(core_axis_name="c", subcore_axis_name="s",
                               num_cores=NC, num_subcores=NS)
_f32 = jnp.float32
_i32 = jnp.int32


def _zero_rows(z64_hbm, shared, s):
    for i in range(RPT // 64):
        pltpu.sync_copy(z64_hbm, shared.at[pl.ds(s * RPT + i * 64, 64)])


def _zero_1d(z1_hbm, shared1, s):
    pltpu.sync_copy(z1_hbm, shared1.at[pl.ds(s * RPT, RPT)])


def _fill_ones(ones):
    for j in range(EC // L):
        ones[pl.ds(j * L, L)] = jnp.full((L,), 1.0, _f32)


@functools.partial(
    pl.kernel,
    out_type=(jax.ShapeDtypeStruct((NWL,), _i32),
              jax.ShapeDtypeStruct((NC, NP), _f32)),
    mesh=_mesh,
    compiler_params=pltpu.CompilerParams(needs_layout_passes=False),
    scratch_types=[
        pltpu.VMEM((N,), _i32),
        pltpu.VMEM((EC,), _i32),
        pltpu.VMEM((EC,), _f32),
        pltpu.VMEM((L,), _i32),
        pltpu.VMEM_SHARED((NP,), _f32),
    ],
)
def _sc_prep(batch_hbm, ed_hbm, z1_hbm, counts_out, deg_out,
             bbuf, edb, ones, crow, dsh):
    c = lax.axis_index("c")
    s = lax.axis_index("s")
    w = c * NS + s
    _zero_1d(z1_hbm, dsh, s)
    _fill_ones(ones)
    pltpu.sync_copy(batch_hbm, bbuf)
    plsc.subcore_barrier()

    g0 = w * 2
    def cbody(j, cc):
        v = bbuf[pl.ds(j * L, L)]
        return (cc[0] + jnp.sum(jnp.where(v == g0, 1, 0)),
                cc[1] + jnp.sum(jnp.where(v == g0 + 1, 1, 0)))
    c0, c1 = lax.fori_loop(0, N // L, cbody, (_i32(0), _i32(0)))
    io = lax.iota(_i32, L)
    crow[...] = jnp.where(io == 0, c0, jnp.where(io == 1, c1, 0))
    pltpu.sync_copy(crow, counts_out.at[pl.ds(w * L, L)])

    base = w * EPW
    def ebody(i, carry):
        pltpu.sync_copy(ed_hbm.at[pl.ds(base + i * EC, EC)], edb)
        pltpu.sync_copy(ones, dsh.at[edb], add=True)
        return carry
    lax.fori_loop(0, EPW // EC, ebody, _i32(0))
    plsc.subcore_barrier()
    pltpu.sync_copy(dsh.at[pl.ds(s * RPT, RPT)],
                    deg_out.at[c, pl.ds(s * RPT, RPT)])


@functools.partial(
    pl.kernel,
    out_type=jax.ShapeDtypeStruct((NC, NP, H), _f32),
    mesh=_mesh,
    compiler_params=pltpu.CompilerParams(needs_layout_passes=False),
    scratch_types=[
        pltpu.VMEM((ECM,), _i32),
        pltpu.VMEM((ECM,), _i32),
        pltpu.VMEM((ECM, H), _f32),
        pltpu.VMEM((L,), _i32),
        pltpu.VMEM_SHARED((NP, H), _f32),
        pltpu.SemaphoreType.DMA,
    ],
)
def _sc_msg(es_hbm, ed_hbm, hs_hbm, ecnt_hbm, z64_hbm, out_hbm,
            esb, edb, rows, erow, fsh, sem):
    c = lax.axis_index("c")
    s = lax.axis_index("s")
    w = c * NS + s
    _zero_rows(z64_hbm, fsh, s)
    pltpu.sync_copy(ecnt_hbm.at[w], erow)
    plsc.subcore_barrier()
    cnt = erow[...][0]
    base = w * EPW
    def body(i, carry):
        pltpu.sync_copy(es_hbm.at[pl.ds(base + i * ECM, ECM)], esb)
        pltpu.sync_copy(ed_hbm.at[pl.ds(base + i * ECM, ECM)], edb)
        pltpu.async_copy(hs_hbm.at[esb], rows, sem).wait()
        pltpu.sync_copy(rows, fsh.at[edb], add=True)
        return carry
    lax.fori_loop(0, cnt // ECM, body, _i32(0))
    plsc.subcore_barrier()
    pltpu.sync_copy(fsh.at[pl.ds(s * RPT, RPT)],
                    out_hbm.at[c, pl.ds(s * RPT, RPT)])


@functools.partial(
    pl.kernel,
    out_type=jax.ShapeDtypeStruct((NC, NP), _f32),
    mesh=_mesh,
    compiler_params=pltpu.CompilerParams(needs_layout_passes=False),
    scratch_types=[
        pltpu.VMEM((EC,), _i32),
        pltpu.VMEM((EC,), _i32),
        pltpu.VMEM((EC,), _f32),
        pltpu.VMEM((L,), _i32),
        pltpu.VMEM_SHARED((NP,), _f32),
        pltpu.SemaphoreType.DMA,
    ],
)
def _sc_smsg(es_hbm, ed_hbm, ss_hbm, ecnt_hbm, z1_hbm, out_hbm,
             esb, edb, svals, erow, ssh, sem):
    c = lax.axis_index("c")
    s = lax.axis_index("s")
    w = c * NS + s
    _zero_1d(z1_hbm, ssh, s)
    pltpu.sync_copy(ecnt_hbm.at[w], erow)
    plsc.subcore_barrier()
    cnt = erow[...][0]
    base = w * EPW
    def body(i, carry):
        pltpu.sync_copy(es_hbm.at[pl.ds(base + i * EC, EC)], esb)
        pltpu.sync_copy(ed_hbm.at[pl.ds(base + i * EC, EC)], edb)
        pltpu.async_copy(ss_hbm.at[esb], svals, sem).wait()
        pltpu.sync_copy(svals, ssh.at[edb], add=True)
        return carry
    lax.fori_loop(0, cnt // EC, body, _i32(0))
    plsc.subcore_barrier()
    pltpu.sync_copy(ssh.at[pl.ds(s * RPT, RPT)],
                    out_hbm.at[c, pl.ds(s * RPT, RPT)])


def _ukeys(kbuf, j):
    x = kbuf[pl.ds(j * L, L)]
    return plsc.bitcast(x, jnp.uint32) ^ jnp.uint32(0x80000000)


def _count_cmp(kbuf, lo, n, t, eq):
    jlo = lo // L
    jhi = (lo + n + L - 1) // L
    def body(j, acc):
        idx = j * L + lax.iota(_i32, L)
        m = (idx >= lo) & (idx < lo + n)
        x = _ukeys(kbuf, j)
        cond = (x == t) if eq else (x >= t)
        return acc + jnp.sum(jnp.where(m & cond, 1, 0))
    return lax.fori_loop(jlo, jhi, body, _i32(0))


def _graph_stats(cbuf, layer, g0):
    def body(r, carry):
        lin, lout, st0, st1 = carry
        v = cbuf[pl.ds(r * L, L)]
        for lane in (0, 1):
            g = 2 * r + lane
            n = v[lane]
            for _ in range(layer - 1):
                n = (n + 1) // 2
            k = (n + 1) // 2
            rec = (lin, n, lout, k)
            st0 = tuple(jnp.where(g == g0, bb, aa)
                        for aa, bb in zip(st0, rec))
            st1 = tuple(jnp.where(g == g0 + 1, bb, aa)
                        for aa, bb in zip(st1, rec))
            lin = lin + n
            lout = lout + k
        return (lin, lout, st0, st1)
    z = (_i32(0), _i32(0), _i32(0), _i32(0))
    _, _, st0, st1 = lax.fori_loop(0, NW, body, (_i32(0), _i32(0), z, z))
    return st0, st1


def _acc_set(t, q, v):
    return t[:q] + (v,) + t[q + 1:]


def _make_sc_pool(layer):
    @functools.partial(
        pl.kernel,
        out_type=(jax.ShapeDtypeStruct((NP, H), _f32),
                  jax.ShapeDtypeStruct((NP,), _i32),
                  jax.ShapeDtypeStruct((B, 2, H), _f32)),
        mesh=_mesh,
        compiler_params=pltpu.CompilerParams(needs_layout_passes=False),
        scratch_types=[
            pltpu.VMEM((NP,), _i32),
            pltpu.VMEM((NWL,), _i32),
            pltpu.VMEM((L, H), _f32),
            pltpu.VMEM((L,), _i32),
            pltpu.VMEM((L,), _i32),
            pltpu.VMEM((L,), _i32),
            pltpu.VMEM((2, H), _f32),
        ],
    )
    def _sc_pool(keys_hbm, gmul_hbm, counts_hbm, hnext_hbm, nmap_hbm, gap_hbm,
                 kbuf, cbuf, rows, sidx, nmi, nmv, gbuf):
        c = lax.axis_index("c")
        s = lax.axis_index("s")
        w = c * NS + s
        pltpu.sync_copy(keys_hbm, kbuf)
        pltpu.sync_copy(counts_hbm, cbuf)
        st0, st1 = _graph_stats(cbuf, layer, w * 2)
        io = lax.iota(_i32, L)

        for gi, st in enumerate((st0, st1)):
            lo, n, lo_out, k = st
            def bit(b, p):
                cand = p | (jnp.uint32(1) << (31 - b).astype(jnp.uint32))
                cnt = _count_cmp(kbuf, lo, n, cand, False)
                return jnp.where(cnt >= k, cand, p)
            thr = lax.fori_loop(0, 32, bit, jnp.uint32(0))
            thr = jnp.where(k > 0, thr, jnp.uint32(0xFFFFFFFF))
            neq = _count_cmp(kbuf, lo, n, thr, True)
            nge = _count_cmp(kbuf, lo, n, thr, False)
            r_allow = k - (nge - neq)

            jlo = lo // L
            jhi = (lo + n + L - 1) // L
            def body(j, carry):
                eqc, kc, accs, accm = carry
                idx = j * L + io
                m = (idx >= lo) & (idx < lo + n)
                x = _ukeys(kbuf, j)
                gt = m & (x > thr)
                eq = m & (x == thr)
                eqi = eq.astype(_i32)
                eqrank = plsc.cumsum(eqi) - eqi + eqc
                keep = gt | (eq & (eqrank < r_allow))
                ki = keep.astype(_i32)
                kpre = plsc.cumsum(ki)
                newid = lo_out + kc + kpre - 1
                sidx[...] = jnp.where(keep, newid, DUMMY + io)
                nmi[...] = jnp.where(m, idx, DUMMY + io)
                nmv[...] = jnp.where(keep, newid, -1)
                pltpu.sync_copy(nmv, nmap_hbm.at[nmi])
                pltpu.sync_copy(gmul_hbm.at[pl.ds(j * L, L)], rows)
                pltpu.sync_copy(rows, hnext_hbm.at[sidx])
                for r in range(L):
                    kr = ki[r] > 0
                    for q in range(H // L):
                        v = rows[r, pl.ds(q * L, L)]
                        accs = _acc_set(accs, q,
                                        accs[q] + jnp.where(kr, v, 0.0))
                        accm = _acc_set(accm, q,
                                        jnp.maximum(accm[q],
                                                    jnp.where(kr, v, -jnp.inf)))
                return (eqc + jnp.sum(eqi), kc + jnp.sum(ki), accs, accm)

            zs = tuple(jnp.zeros((L,), _f32) for _ in range(H // L))
            zm = tuple(jnp.full((L,), -jnp.inf, _f32) for _ in range(H // L))
            _, _, accs, accm = lax.fori_loop(jlo, jhi, body,
                                             (_i32(0), _i32(0), zs, zm))
            kf = jnp.maximum(k, 1).astype(_f32)
            for q in range(H // L):
                gbuf[0, pl.ds(q * L, L)] = jnp.where(k > 0, accm[q], 0.0)
                gbuf[1, pl.ds(q * L, L)] = accs[q] / kf
            pltpu.sync_copy(gbuf, gap_hbm.at[w * 2 + gi])

    return _sc_pool


_SC_POOL = {l: _make_sc_pool(l) for l in (1, 2, 3)}


@functools.partial(
    pl.kernel,
    out_type=(jax.ShapeDtypeStruct((E,), _i32),
              jax.ShapeDtypeStruct((E,), _i32),
              jax.ShapeDtypeStruct((NW, L), _i32),
              jax.ShapeDtypeStruct((NC, NP), _f32)),
    mesh=_mesh,
    compiler_params=pltpu.CompilerParams(needs_layout_passes=False),
    scratch_types=[
        pltpu.VMEM((NP,), _i32),
        pltpu.VMEM((EC,), _i32),
        pltpu.VMEM((EC,), _i32),
        pltpu.VMEM((EPW,), _i32),
        pltpu.VMEM((EPW,), _i32),
        pltpu.VMEM((EC,), _i32),
        pltpu.VMEM((EC,), _f32),
        pltpu.VMEM((L,), _i32),
        pltpu.VMEM_SHARED((NP,), _f32),
    ],
)
def _sc_remap(es_hbm, ed_hbm, nmap_hbm, ecnt_hbm, z1_hbm,
              es_out, ed_out, ecnt_out, deg_out,
              nmb, esb, edb, oes, oed, dstage, ones, erow, dsh):
    c = lax.axis_index("c")
    s = lax.axis_index("s")
    w = c * NS + s
    _zero_1d(z1_hbm, dsh, s)
    _fill_ones(ones)
    pltpu.sync_copy(nmap_hbm, nmb)
    pltpu.sync_copy(ecnt_hbm.at[w], erow)
    plsc.subcore_barrier()
    cnt_in = erow[...][0]
    base = w * EPW
    io = lax.iota(_i32, L)

    def body(i, nv):
        pltpu.sync_copy(es_hbm.at[pl.ds(base + i * EC, EC)], esb)
        pltpu.sync_copy(ed_hbm.at[pl.ds(base + i * EC, EC)], edb)
        for j in range(EC // L):
            sv = esb[pl.ds(j * L, L)]
            dv = edb[pl.ds(j * L, L)]
            ms = plsc.load_gather(nmb, [sv])
            md = plsc.load_gather(nmb, [dv])
            valid = (ms >= 0) & (md >= 0)
            vi = valid.astype(_i32)
            pos = nv + plsc.cumsum(vi) - 1
            plsc.store_scatter(oes, [pos], ms, mask=valid)
            plsc.store_scatter(oed, [pos], md, mask=valid)
            nv = nv + jnp.sum(vi)
        return nv
    nv = lax.fori_loop(0, cnt_in // EC, body, _i32(0))
    cnt_pad = ((nv + EC - 1) // EC) * EC

    def pbody(j, carry):
        idx = j * L + io
        pad = idx >= nv
        v1 = oes[pl.ds(j * L, L)]
        v2 = oed[pl.ds(j * L, L)]
        oes[pl.ds(j * L, L)] = jnp.where(pad, io, v1)
        oed[pl.ds(j * L, L)] = jnp.where(pad, DUMMY + io, v2)
        return carry
    lax.fori_loop(nv // L, cnt_pad // L, pbody, _i32(0))

    def obody(i, carry):
        pltpu.sync_copy(oes.at[pl.ds(i * EC, EC)],
                        es_out.at[pl.ds(base + i * EC, EC)])
        pltpu.sync_copy(oed.at[pl.ds(i * EC, EC)],
                        ed_out.at[pl.ds(base + i * EC, EC)])
        for j in range(EC // L):
            dstage[pl.ds(j * L, L)] = oed[pl.ds(i * EC + j * L, L)]
        pltpu.sync_copy(ones, dsh.at[dstage], add=True)
        return carry
    lax.fori_loop(0, cnt_pad // EC, obody, _i32(0))
    erow[...] = jnp.where(io == 0, cnt_pad, 0)
    pltpu.sync_copy(erow, ecnt_out.at[w])
    plsc.subcore_barrier()
    pltpu.sync_copy(dsh.at[pl.ds(s * RPT, RPT)],
                    deg_out.at[c, pl.ds(s * RPT, RPT)])


def _counts_k(cmat, layer):
    c = cmat
    for _ in range(layer - 1):
        c = (c + 1) // 2
    return c


def _tc_scale_body(layer, h_ref, w_ref, deg_ref, cmat_ref, hs_ref, dinv_ref):
    kvalid = jnp.sum(_counts_k(cmat_ref[...], layer))
    deg = deg_ref[0, :, :] + deg_ref[1, :, :]
    row = lax.broadcasted_iota(_i32, (NP, 1), 0)
    deg = deg + jnp.where(row < kvalid, 1.0, 0.0)
    dinv = jnp.where(deg > 0, lax.rsqrt(deg), 0.0)
    hw = jnp.dot(h_ref[...], w_ref[...], preferred_element_type=_f32)
    hs_ref[...] = hw * dinv
    dinv_ref[...] = dinv


def _tc_scale(layer, h, w, deg, cmat):
    return pl.pallas_call(
        functools.partial(_tc_scale_body, layer),
        out_shape=(jax.ShapeDtypeStruct((NP, H), _f32),
                   jax.ShapeDtypeStruct((NP, 1), _f32)),
    )(h, w, deg.reshape(NC, NP, 1), cmat)


def _tc_update_body(msg_ref, hs_ref, dinv_ref, b_ref, ws_ref, h1_ref, ss_ref):
    dinv = dinv_ref[...]
    m = (msg_ref[0, :, :] + msg_ref[1, :, :]) * dinv + hs_ref[...] * dinv
    h1 = jnp.maximum(m + b_ref[...], 0.0)
    h1_ref[...] = h1
    sp = jnp.dot(h1, ws_ref[...], preferred_element_type=_f32)
    ss_ref[...] = sp[:, 0:1] * dinv


def _tc_update(msg, hs, dinv, b, ws_pad):
    return pl.pallas_call(
        _tc_update_body,
        out_shape=(jax.ShapeDtypeStruct((NP, H), _f32),
                   jax.ShapeDtypeStruct((NP, 1), _f32)),
    )(msg, hs, dinv, b.reshape(1, H), ws_pad)


def _tc_score_body(layer, smsg_ref, ss_ref, dinv_ref, bs_ref, h1_ref,
                   cmat_ref, keys_ref, gmul_ref):
    dinv = dinv_ref[...]
    score = (smsg_ref[0, :, :] + smsg_ref[1, :, :]) * dinv \
        + ss_ref[...] * dinv + bs_ref[0, 0]
    kvalid = jnp.sum(_counts_k(cmat_ref[...], layer))
    row = lax.broadcasted_iota(_i32, (NP, 1), 0)
    bits = lax.bitcast_convert_type(score, _i32)
    key = bits ^ ((bits >> 31) & jnp.int32(0x7FFFFFFF))
    keys_ref[...] = jnp.where(row < kvalid, key, NEG)
    gmul_ref[...] = h1_ref[...] * jnp.tanh(score)


def _tc_score(layer, smsg, ss, dinv, bs, h1, cmat):
    return pl.pallas_call(
        functools.partial(_tc_score_body, layer),
        out_shape=(jax.ShapeDtypeStruct((NP, 1), _i32),
                   jax.ShapeDtypeStruct((NP, H), _f32)),
    )(smsg.reshape(NC, NP, 1), ss, dinv, bs.reshape(1, 1), h1, cmat)


def _tc_final_body(g1_ref, g2_ref, g3_ref,
                   l1w_ref, l1b_ref, l2w_ref, l2b_ref, l3w_ref, l3b_ref,
                   out_ref):
    z = jnp.zeros((B, 2 * H), _f32)
    for g_ref in (g1_ref, g2_ref, g3_ref):
        z = z + jnp.concatenate([g_ref[:, 0, :], g_ref[:, 1, :]], axis=1)
    z = jnp.maximum(jnp.dot(z, l1w_ref[...], preferred_element_type=_f32)
                    + l1b_ref[...], 0.0)
    z = jnp.maximum(jnp.dot(z, l2w_ref[...], preferred_element_type=_f32)
                    + l2b_ref[...], 0.0)
    z = jnp.dot(z, l3w_ref[...], preferred_element_type=_f32) + l3b_ref[...]
    z = z[:, 0:10]
    m = jnp.max(z, axis=1, keepdims=True)
    lse = m + jnp.log(jnp.sum(jnp.exp(z - m), axis=1, keepdims=True))
    out_ref[...] = z - lse


def _tc_final(g1, g2, g3, l1w, l1b, l2w, l2b, l3w, l3b):
    return pl.pallas_call(
        _tc_final_body,
        out_shape=jax.ShapeDtypeStruct((B, 10), _f32),
    )(g1, g2, g3, l1w, l1b, l2w, l2b, l3w, l3b)


def kernel(x, edge_index, batch, W1, b1, Ws1, bs1, W2, b2, Ws2, bs2,
           W3, b3, Ws3, bs3, L1w, L1b, L2w, L2b, L3w, L3b):
    es = edge_index[0]
    ed = edge_index[1]
    z1 = jnp.zeros((RPT,), _f32)
    z64 = jnp.zeros((64, H), _f32)
    ecnt = jnp.concatenate([jnp.full((NW, 1), EPW, _i32),
                            jnp.zeros((NW, L - 1), _i32)], axis=1)

    cmat, deg = _sc_prep(batch, ed, z1)
    cmat_tc = cmat.reshape(1, NWL)

    h = jnp.pad(x, ((0, NP - N), (0, 0)))
    gaps = []
    for layer, (W, b, Ws, bs) in enumerate(
            ((W1, b1, Ws1, bs1), (W2, b2, Ws2, bs2), (W3, b3, Ws3, bs3)),
            start=1):
        ws_pad = jnp.pad(Ws, ((0, 0), (0, H - 1)))
        hs, dinv = _tc_scale(layer, h, W, deg, cmat_tc)
        msg = _sc_msg(es, ed, hs, ecnt, z64)
        h1, ss = _tc_update(msg, hs, dinv, b, ws_pad)
        smsg = _sc_smsg(es, ed, ss.reshape(NP), ecnt, z1)
        keys, gmul = _tc_score(layer, smsg, ss, dinv, bs, h1, cmat_tc)
        hnext, nmap, gap = _SC_POOL[layer](keys.reshape(NP), gmul, cmat)
        gaps.append(gap)
        if layer < 3:
            es, ed, ecnt, deg = _sc_remap(es, ed, nmap, ecnt, z1)
        h = hnext

    l2w_pad = jnp.pad(L2w, ((0, 0), (0, H - L2w.shape[1])))
    l3w_pad = jnp.pad(L3w, ((0, H - L3w.shape[0]), (0, H - L3w.shape[1])))
    l2b_pad = jnp.pad(L2b, (0, H - L2b.shape[0])).reshape(1, H)
    l3b_pad = jnp.pad(L3b, (0, H - L3b.shape[0])).reshape(1, H)
    return _tc_final(gaps[0], gaps[1], gaps[2],
                     L1w, L1b.reshape(1, H), l2w_pad, l2b_pad,
                     l3w_pad, l3b_pad)
